# Initial kernel scaffold; baseline (speedup 1.0000x reference)
import jax
import jax.numpy as jnp
from jax.experimental import pallas as pl


def kernel(feat, delay, value, edge_index, node_level, po_idx, W_pi1, b_pi1, W_pi2, b_pi2, W_agg1, b_agg1, W_agg2, b_agg2, W_out1, b_out1, W_out2, b_out2):
    raise NotImplementedError("write your pallas kernel here")


# speedup vs baseline: 8.7967x; 8.7967x over previous
import functools

import jax
import jax.numpy as jnp
from jax import lax
from jax.experimental import pallas as pl
from jax.experimental.pallas import tpu as pltpu
from jax.experimental.pallas import tpu_sc as plsc

N = 10000
E = 320000
L = 8
INFEAT = 128
HID = 256
HALF = 128
P = 2048

NP = 10240
BLK = 1280
NBLK = NP // BLK
WG = HALF

NC = 2
NS = 16
NW = NC * NS
PER_W = E // NW
CH = 200
NCH = PER_W // CH

ROWS_PER_SUB = NP // NS

_mesh = plsc.VectorSubcoreMesh(core_axis_name="c", subcore_axis_name="s")


@functools.partial(
    pl.kernel,
    mesh=_mesh,
    out_type=jax.ShapeDtypeStruct((2 * NP, WG), jnp.float32),
    scratch_types=[
        pltpu.VMEM((CH,), jnp.int32),
        pltpu.VMEM((CH,), jnp.int32),
        pltpu.VMEM((CH, WG), jnp.float32),
        pltpu.VMEM_SHARED((NP, WG), jnp.float32),
        pltpu.SemaphoreType.DMA,
    ],
)
def _sc_scatter(g_hbm, src_hbm, dst_hbm, zeros_hbm, out_hbm,
                src_v, dst_v, rows_v, s_sh, sem):
    cid = lax.axis_index("c")
    sid = lax.axis_index("s")
    r0 = sid * ROWS_PER_SUB
    pltpu.sync_copy(zeros_hbm.at[pl.ds(r0, ROWS_PER_SUB)],
                    s_sh.at[pl.ds(r0, ROWS_PER_SUB)])
    plsc.subcore_barrier()
    wid = sid * NC + cid
    base = wid * PER_W
    for k in range(NCH):
        off = base + k * CH
        pltpu.sync_copy(src_hbm.at[pl.ds(off, CH)], src_v)
        pltpu.sync_copy(dst_hbm.at[pl.ds(off, CH)], dst_v)
        pltpu.async_copy(g_hbm.at[src_v], rows_v, sem).wait()
        pltpu.sync_copy(rows_v, s_sh.at[dst_v], add=True)
    plsc.subcore_barrier()
    pltpu.sync_copy(s_sh.at[pl.ds(r0, ROWS_PER_SUB)],
                    out_hbm.at[pl.ds(cid * NP + r0, ROWS_PER_SUB)])


PO_PER_W = P // NW


@functools.partial(
    pl.kernel,
    mesh=_mesh,
    out_type=jax.ShapeDtypeStruct((P, HID), jnp.float32),
    scratch_types=[
        pltpu.VMEM((PO_PER_W,), jnp.int32),
        pltpu.VMEM((PO_PER_W, HID), jnp.float32),
        pltpu.SemaphoreType.DMA,
    ],
)
def _sc_gather_po(h_hbm, po_hbm, out_hbm, idx_v, rows_v, sem):
    cid = lax.axis_index("c")
    sid = lax.axis_index("s")
    base = (sid * NC + cid) * PO_PER_W
    pltpu.sync_copy(po_hbm.at[pl.ds(base, PO_PER_W)], idx_v)
    pltpu.async_copy(h_hbm.at[idx_v], rows_v, sem).wait()
    pltpu.sync_copy(rows_v, out_hbm.at[pl.ds(base, PO_PER_W)])


def _leaky(x):
    return jnp.where(x > 0, x, 0.1 * x)


def _init_body(x4, feat, nl, wpi1, bpi1, wpi2, bpi2, wbot, bagg1, wtop,
               h0_o, gaug_o, fw_o):
    t = _leaky(jnp.dot(x4[...], wpi1[...], preferred_element_type=jnp.float32)
               + bpi1[...])
    hpi = jnp.dot(t, wpi2[...], preferred_element_type=jnp.float32) + bpi2[...]
    m = nl[...][:, :1] == 0
    h0 = jnp.where(m, hpi, 0.0)
    h0_o[...] = h0
    fw_o[...] = (jnp.dot(feat[...], wbot[...],
                         preferred_element_type=jnp.float32) + bagg1[...])
    gaug_o[...] = jnp.dot(h0, wtop[...], preferred_element_type=jnp.float32)


def _level_body(lvl, s0, s1, d0, d1, fw, nl, h, gaug, wagg2, bagg2, wtop,
                h_o, gaug_o):
    deg = d0[...][:, :1] + d1[...][:, :1]
    inv = 1.0 / jnp.maximum(deg, 1.0)
    pre = (s0[...][:, :HALF] + s1[...][:, :HALF]) * inv + fw[...]
    act = _leaky(pre)
    hh = jnp.dot(act, wagg2[...], preferred_element_type=jnp.float32) + bagg2[...]
    m = nl[...][:, :1] == lvl
    h_new = jnp.where(m, hh, h[...])
    h_o[...] = h_new
    gaug_o[...] = jnp.where(m, jnp.dot(hh, wtop[...],
                                       preferred_element_type=jnp.float32),
                            gaug[...])


def _out_body(hpo, w1, b1, w2, b2, o):
    t = _leaky(jnp.dot(hpo[...], w1[...], preferred_element_type=jnp.float32)
               + b1[...])
    o[...] = jnp.dot(t, w2[...], preferred_element_type=jnp.float32) + b2[...]


def _row_spec(d):
    return pl.BlockSpec((BLK, d), lambda i: (i, 0))


def _full_spec(r, c):
    return pl.BlockSpec((r, c), lambda i: (0, 0))


def kernel(feat, delay, value, edge_index, node_level, po_idx,
           W_pi1, b_pi1, W_pi2, b_pi2,
           W_agg1, b_agg1, W_agg2, b_agg2,
           W_out1, b_out1, W_out2, b_out2):
    f32 = jnp.float32
    pad = NP - N
    featp = jnp.pad(feat, ((0, pad), (0, 0)))
    x4 = jnp.pad(jnp.concatenate([delay, value], axis=1), ((0, pad), (0, 0)))
    nl = jnp.pad(node_level.astype(jnp.int32), (0, pad), constant_values=127)
    nl_b = jnp.broadcast_to(nl[:, None], (NP, 8))
    src = edge_index[0].astype(jnp.int32)
    dst = edge_index[1].astype(jnp.int32)
    po = po_idx.astype(jnp.int32)
    zeros = jnp.zeros((NP, WG), f32)
    wtop = W_agg1[:HID]
    wbot = W_agg1[HID:]
    bpi1 = b_pi1.reshape(1, HALF)
    bpi2 = b_pi2.reshape(1, HID)
    bagg1 = b_agg1.reshape(1, HALF)
    bagg2 = b_agg2.reshape(1, HID)
    bout1 = b_out1.reshape(1, HID)
    bout2 = b_out2.reshape(1, 1)

    h, gaug, fw = pl.pallas_call(
        _init_body,
        grid=(NBLK,),
        in_specs=[
            _row_spec(4), _row_spec(INFEAT), _row_spec(8),
            _full_spec(4, HALF), _full_spec(1, HALF),
            _full_spec(HALF, HID), _full_spec(1, HID),
            _full_spec(INFEAT, HALF), _full_spec(1, HALF),
            _full_spec(HID, HALF),
        ],
        out_specs=[_row_spec(HID), _row_spec(WG), _row_spec(HALF)],
        out_shape=[
            jax.ShapeDtypeStruct((NP, HID), f32),
            jax.ShapeDtypeStruct((NP, WG), f32),
            jax.ShapeDtypeStruct((NP, HALF), f32),
        ],
    )(x4, featp, nl_b, W_pi1, bpi1, W_pi2, bpi2, wbot, bagg1, wtop)

    ones_tab = jnp.ones((NP, WG), f32)
    d_all = _sc_scatter(ones_tab, src, dst, zeros)

    for lvl in range(1, L):
        s_all = _sc_scatter(gaug, src, dst, zeros)
        h, gaug = pl.pallas_call(
            functools.partial(_level_body, lvl),
            grid=(NBLK,),
            in_specs=[
                pl.BlockSpec((BLK, WG), lambda i: (i, 0)),
                pl.BlockSpec((BLK, WG), lambda i: (i + NBLK, 0)),
                pl.BlockSpec((BLK, WG), lambda i: (i, 0)),
                pl.BlockSpec((BLK, WG), lambda i: (i + NBLK, 0)),
                _row_spec(HALF), _row_spec(8), _row_spec(HID),
                _row_spec(WG),
                _full_spec(HALF, HID), _full_spec(1, HID),
                _full_spec(HID, HALF),
            ],
            out_specs=[_row_spec(HID), _row_spec(WG)],
            out_shape=[
                jax.ShapeDtypeStruct((NP, HID), f32),
                jax.ShapeDtypeStruct((NP, WG), f32),
            ],
        )(s_all, s_all, d_all, d_all, fw, nl_b, h, gaug, W_agg2, bagg2, wtop)

    h_po = _sc_gather_po(h, po)

    rst = pl.pallas_call(
        _out_body,
        grid=(1,),
        in_specs=[
            pl.BlockSpec((P, HID), lambda i: (0, 0)),
            _full_spec(HID, HID), _full_spec(1, HID),
            _full_spec(HID, 1), _full_spec(1, 1),
        ],
        out_specs=pl.BlockSpec((P, 1), lambda i: (0, 0)),
        out_shape=jax.ShapeDtypeStruct((P, 1), f32),
    )(h_po, W_out1, bout1, W_out2, bout2)
    return rst
